# Initial kernel scaffold; baseline (speedup 1.0000x reference)
import jax
import jax.numpy as jnp
from jax.experimental import pallas as pl

_K = 64
_N = 32768
_ROWS = 128
_BLOCK_ROWS = 8


def _topk_mask_body(x_ref, o_ref):
    x = x_ref[...]
    r = x.shape[0]
    xi = jax.lax.bitcast_convert_type(x, jnp.int32)
    z = xi ^ ((xi >> 31) & jnp.int32(0x7FFFFFFF))

    def count_ge(t):
        return jnp.sum((z >= t).astype(jnp.float32), axis=1, keepdims=True)

    kf = jnp.float32(_K)
    one = jnp.full((r, 1), 1, jnp.int32)
    c1 = count_ge(one)
    zmax = jnp.max(z, axis=1, keepdims=True)

    done0 = c1 <= kf
    lo0 = one
    hi0 = jnp.maximum(zmax + 1, one + 1)

    def cond(state):
        lo, hi, done = state
        return jnp.any(jnp.logical_not(done))

    def body(state):
        lo, hi, done = state
        mid = lo + (hi - lo) // 2
        c = count_ge(mid)
        not_done = jnp.logical_not(done)
        go_lo = jnp.logical_and(not_done, c >= kf)
        go_hi = jnp.logical_and(not_done, c < kf)
        lo = jnp.where(go_lo, mid, lo)
        hi = jnp.where(go_hi, mid, hi)
        done = jnp.logical_or(done, jnp.logical_or(c == kf, hi - lo <= 1))
        return lo, hi, done

    lo, _, _ = jax.lax.while_loop(cond, body, (lo0, hi0, done0))

    cnt = count_ge(lo)
    ties = jnp.any(cnt > kf)

    @pl.when(jnp.logical_not(ties))
    def _():
        o_ref[...] = jnp.where(z >= lo, x, jnp.float32(0.0))

    @pl.when(ties)
    def _():
        surplus = cnt > kf
        cgt = count_ge(lo + 1)
        want = jnp.where(surplus, kf - cgt, jnp.float32(_N))
        tie = jnp.logical_and(z == lo, surplus)
        col = jax.lax.broadcasted_iota(jnp.int32, (r, _N), 1)

        def tcount(j):
            m = jnp.logical_and(tie, col < j)
            return jnp.sum(m.astype(jnp.float32), axis=1, keepdims=True)

        def tbody(i, st):
            tlo, thi = st
            mid = tlo + (thi - tlo) // 2
            c = tcount(mid)
            small = c <= want
            tlo = jnp.where(small, mid, tlo)
            thi = jnp.where(small, thi, mid)
            return tlo, thi

        jlo0 = jnp.zeros((r, 1), jnp.int32)
        jhi0 = jnp.full((r, 1), _N + 1, jnp.int32)
        jcut, _ = jax.lax.fori_loop(0, 16, tbody, (jlo0, jhi0))

        keep = jnp.where(surplus,
                         jnp.logical_or(z >= lo + 1,
                                        jnp.logical_and(tie, col < jcut)),
                         z >= lo)
        o_ref[...] = jnp.where(keep, x, jnp.float32(0.0))


def kernel(x):
    grid = _ROWS // _BLOCK_ROWS
    return pl.pallas_call(
        _topk_mask_body,
        grid=(grid,),
        in_specs=[pl.BlockSpec((_BLOCK_ROWS, _N), lambda i: (i, 0))],
        out_specs=pl.BlockSpec((_BLOCK_ROWS, _N), lambda i: (i, 0)),
        out_shape=jax.ShapeDtypeStruct((_ROWS, _N), jnp.float32),
    )(x)


# speedup vs baseline: 6.3664x; 6.3664x over previous
import jax
import jax.numpy as jnp
from jax.experimental import pallas as pl

_K = 64
_N = 32768
_ROWS = 128
_BLOCK_ROWS = 8


def _topk_mask_body(x_ref, o_ref):
    x = x_ref[...]
    r = x.shape[0]
    xi = jax.lax.bitcast_convert_type(x, jnp.int32)
    z = xi ^ ((xi >> 31) & jnp.int32(0x7FFFFFFF))

    def count_ge(t):
        ind = jnp.where(z >= t, jnp.float32(1.0), jnp.float32(0.0))
        return jnp.sum(ind, axis=1, keepdims=True)

    kf = jnp.float32(_K)
    one = jnp.full((r, 1), 1, jnp.int32)
    c1 = count_ge(one)
    zmax = jnp.max(z, axis=1, keepdims=True)

    done0 = jnp.where(c1 <= kf, jnp.int32(1), jnp.int32(0))
    lo0 = one
    hi0 = jnp.maximum(zmax + 1, one + 1)

    def cond(state):
        lo, hi, done = state
        return jnp.min(done) < 1

    def body(state):
        lo, hi, done = state
        mid = lo + (hi - lo) // 2
        c = count_ge(mid)
        not_done = done < 1
        go_lo = jnp.logical_and(not_done, c >= kf)
        go_hi = jnp.logical_and(not_done, c < kf)
        lo = jnp.where(go_lo, mid, lo)
        hi = jnp.where(go_hi, mid, hi)
        fin = jnp.logical_or(c == kf, hi - lo <= 1)
        done = jnp.maximum(done, jnp.where(fin, jnp.int32(1), jnp.int32(0)))
        return lo, hi, done

    lo, _, _ = jax.lax.while_loop(cond, body, (lo0, hi0, done0))

    cnt = count_ge(lo)
    ties = jnp.any(cnt > kf)

    @pl.when(jnp.logical_not(ties))
    def _():
        o_ref[...] = jnp.where(z >= lo, x, jnp.float32(0.0))

    @pl.when(ties)
    def _():
        surplus = cnt > kf
        cgt = count_ge(lo + 1)
        want = jnp.where(surplus, kf - cgt, jnp.float32(_N))
        tie = jnp.logical_and(z == lo, surplus)
        col = jax.lax.broadcasted_iota(jnp.int32, (r, _N), 1)

        def tcount(j):
            m = jnp.logical_and(tie, col < j)
            ind = jnp.where(m, jnp.float32(1.0), jnp.float32(0.0))
            return jnp.sum(ind, axis=1, keepdims=True)

        def tbody(i, st):
            tlo, thi = st
            mid = tlo + (thi - tlo) // 2
            c = tcount(mid)
            small = c <= want
            tlo = jnp.where(small, mid, tlo)
            thi = jnp.where(small, thi, mid)
            return tlo, thi

        jlo0 = jnp.zeros((r, 1), jnp.int32)
        jhi0 = jnp.full((r, 1), _N + 1, jnp.int32)
        jcut, _ = jax.lax.fori_loop(0, 16, tbody, (jlo0, jhi0))

        ok_tie = jnp.logical_or(z >= lo + 1,
                                jnp.logical_and(tie, col < jcut))
        keep = jnp.logical_or(jnp.logical_and(surplus, ok_tie),
                              jnp.logical_and(jnp.logical_not(surplus),
                                              z >= lo))
        o_ref[...] = jnp.where(keep, x, jnp.float32(0.0))


def kernel(x):
    grid = _ROWS // _BLOCK_ROWS
    return pl.pallas_call(
        _topk_mask_body,
        grid=(grid,),
        in_specs=[pl.BlockSpec((_BLOCK_ROWS, _N), lambda i: (i, 0))],
        out_specs=pl.BlockSpec((_BLOCK_ROWS, _N), lambda i: (i, 0)),
        out_shape=jax.ShapeDtypeStruct((_ROWS, _N), jnp.float32),
    )(x)


# speedup vs baseline: 7.1353x; 1.1208x over previous
import jax
import jax.numpy as jnp
from jax.experimental import pallas as pl

_K = 64
_N = 32768
_ROWS = 128
_BLOCK_ROWS = 8


def _topk_mask_body(x_ref, o_ref):
    x = x_ref[...]
    r = x.shape[0]
    xi = jax.lax.bitcast_convert_type(x, jnp.int32)
    z = xi ^ ((xi >> 31) & jnp.int32(0x7FFFFFFF))

    def count_ge(t):
        ind = jnp.where(z >= t, jnp.float32(1.0), jnp.float32(0.0))
        return jnp.sum(ind, axis=1, keepdims=True)

    kf = jnp.float32(_K)
    one = jnp.full((r, 1), 1, jnp.int32)
    c1 = count_ge(one)

    zc = jnp.max(z.reshape(r, _N // 128, 128), axis=1)
    zmax = jnp.max(zc, axis=1, keepdims=True)

    def countc_ge(t):
        ind = jnp.where(zc >= t, jnp.float32(1.0), jnp.float32(0.0))
        return jnp.sum(ind, axis=1, keepdims=True)

    hi0 = jnp.maximum(zmax + 1, one + 1)

    def s1body(i, st):
        lo, hi = st
        mid = lo + (hi - lo) // 2
        cc = countc_ge(mid)
        big = cc >= kf
        lo = jnp.where(big, mid, lo)
        hi = jnp.where(big, hi, mid)
        return lo, hi

    s1lo, _ = jax.lax.fori_loop(0, 22, s1body, (one, hi0))

    done0 = jnp.where(c1 <= kf, jnp.int32(1), jnp.int32(0))
    lo0 = jnp.maximum(s1lo, one)

    def cond(state):
        lo, hi, done = state
        return jnp.min(done) < 1

    def body(state):
        lo, hi, done = state
        mid = lo + (hi - lo) // 2
        c = count_ge(mid)
        not_done = done < 1
        go_lo = jnp.logical_and(not_done, c >= kf)
        go_hi = jnp.logical_and(not_done, c < kf)
        lo = jnp.where(go_lo, mid, lo)
        hi = jnp.where(go_hi, mid, hi)
        fin = jnp.logical_or(c == kf, hi - lo <= 1)
        done = jnp.maximum(done, jnp.where(fin, jnp.int32(1), jnp.int32(0)))
        return lo, hi, done

    lo, _, _ = jax.lax.while_loop(cond, body, (lo0, hi0, done0))

    ind = jnp.where(z >= lo, jnp.float32(1.0), jnp.float32(0.0))
    cnt = jnp.sum(ind, axis=1, keepdims=True)
    o_ref[...] = x * ind
    ties = jnp.any(cnt > kf)

    @pl.when(ties)
    def _():
        surplus = cnt > kf
        cgt = count_ge(lo + 1)
        want = jnp.where(surplus, kf - cgt, jnp.float32(_N))
        tie = jnp.logical_and(z == lo, surplus)
        col = jax.lax.broadcasted_iota(jnp.int32, (r, _N), 1)

        def tcount(j):
            m = jnp.logical_and(tie, col < j)
            ind = jnp.where(m, jnp.float32(1.0), jnp.float32(0.0))
            return jnp.sum(ind, axis=1, keepdims=True)

        def tbody(i, st):
            tlo, thi = st
            mid = tlo + (thi - tlo) // 2
            c = tcount(mid)
            small = c <= want
            tlo = jnp.where(small, mid, tlo)
            thi = jnp.where(small, thi, mid)
            return tlo, thi

        jlo0 = jnp.zeros((r, 1), jnp.int32)
        jhi0 = jnp.full((r, 1), _N + 1, jnp.int32)
        jcut, _ = jax.lax.fori_loop(0, 16, tbody, (jlo0, jhi0))

        ok_tie = jnp.logical_or(z >= lo + 1,
                                jnp.logical_and(tie, col < jcut))
        keep = jnp.logical_or(jnp.logical_and(surplus, ok_tie),
                              jnp.logical_and(jnp.logical_not(surplus),
                                              z >= lo))
        o_ref[...] = jnp.where(keep, x, jnp.float32(0.0))


def kernel(x):
    grid = _ROWS // _BLOCK_ROWS
    return pl.pallas_call(
        _topk_mask_body,
        grid=(grid,),
        in_specs=[pl.BlockSpec((_BLOCK_ROWS, _N), lambda i: (i, 0))],
        out_specs=pl.BlockSpec((_BLOCK_ROWS, _N), lambda i: (i, 0)),
        out_shape=jax.ShapeDtypeStruct((_ROWS, _N), jnp.float32),
    )(x)


# speedup vs baseline: 10.0764x; 1.4122x over previous
import jax
import jax.numpy as jnp
from jax.experimental import pallas as pl

_K = 64
_N = 32768
_ROWS = 128
_BLOCK_ROWS = 8


def _topk_mask_body(x_ref, o_ref):
    x = x_ref[...]
    r = x.shape[0]
    xi = jax.lax.bitcast_convert_type(x, jnp.int32)
    z = xi ^ ((xi >> 31) & jnp.int32(0x7FFFFFFF))

    def count_ge(t):
        ind = jnp.where(z >= t, jnp.float32(1.0), jnp.float32(0.0))
        return jnp.sum(ind, axis=1, keepdims=True)

    kf = jnp.float32(_K)
    one = jnp.full((r, 1), 1, jnp.int32)
    c1 = count_ge(one)

    zc = jnp.max(z.reshape(r, _N // 128, 128), axis=1)
    zmax = jnp.max(zc, axis=1, keepdims=True)

    def countc_ge(t):
        ind = jnp.where(zc >= t, jnp.float32(1.0), jnp.float32(0.0))
        return jnp.sum(ind, axis=1, keepdims=True)

    hi0 = jnp.maximum(zmax + 1, one + 1)

    def s1body(i, st):
        lo, hi = st
        q = (hi - lo) // 4
        m1 = lo + q
        m2 = lo + 2 * q
        m3 = lo + 3 * q
        m2 = jnp.maximum(m2, lo + 1)
        m3 = jnp.maximum(m3, m2)
        cc1 = countc_ge(m1)
        cc2 = countc_ge(m2)
        cc3 = countc_ge(m3)
        lo = jnp.where(cc1 >= kf, m1, lo)
        lo = jnp.where(cc2 >= kf, m2, lo)
        lo = jnp.where(cc3 >= kf, m3, lo)
        hi = jnp.where(cc3 < kf, m3, hi)
        hi = jnp.where(cc2 < kf, m2, hi)
        hi = jnp.where(cc1 < kf, m1, hi)
        return lo, hi

    s1lo, _ = jax.lax.fori_loop(0, 12, s1body, (one, hi0))

    done0 = jnp.where(c1 <= kf, jnp.int32(1), jnp.int32(0))
    lo0 = jnp.maximum(s1lo, one)

    def cond(state):
        lo, hi, done = state
        return jnp.min(done) < 1

    def body(state):
        lo, hi, done = state
        q = (hi - lo) // 4
        m1 = lo + q
        m2 = lo + 2 * q
        m3 = lo + 3 * q
        m2 = jnp.maximum(m2, lo + 1)
        m3 = jnp.maximum(m3, m2)
        c1_ = count_ge(m1)
        c2_ = count_ge(m2)
        c3_ = count_ge(m3)
        not_done = done < 1
        lo = jnp.where(jnp.logical_and(not_done, c1_ >= kf), m1, lo)
        lo = jnp.where(jnp.logical_and(not_done, c2_ >= kf), m2, lo)
        lo = jnp.where(jnp.logical_and(not_done, c3_ >= kf), m3, lo)
        hi = jnp.where(jnp.logical_and(not_done, c3_ < kf), m3, hi)
        hi = jnp.where(jnp.logical_and(not_done, c2_ < kf), m2, hi)
        hi = jnp.where(jnp.logical_and(not_done, c1_ < kf), m1, hi)
        lo = jnp.where(jnp.logical_and(not_done, c1_ == kf), m1, lo)
        lo = jnp.where(jnp.logical_and(not_done, c2_ == kf), m2, lo)
        lo = jnp.where(jnp.logical_and(not_done, c3_ == kf), m3, lo)
        hit = jnp.logical_or(jnp.logical_or(c1_ == kf, c2_ == kf), c3_ == kf)
        fin = jnp.logical_or(hit, hi - lo <= 1)
        done = jnp.maximum(done, jnp.where(fin, jnp.int32(1), jnp.int32(0)))
        return lo, hi, done

    lo, _, _ = jax.lax.while_loop(cond, body, (lo0, hi0, done0))

    ind = jnp.where(z >= lo, jnp.float32(1.0), jnp.float32(0.0))
    cnt = jnp.sum(ind, axis=1, keepdims=True)
    o_ref[...] = x * ind
    ties = jnp.any(cnt > kf)

    @pl.when(ties)
    def _():
        surplus = cnt > kf
        cgt = count_ge(lo + 1)
        want = jnp.where(surplus, kf - cgt, jnp.float32(_N))
        tie = jnp.logical_and(z == lo, surplus)
        col = jax.lax.broadcasted_iota(jnp.int32, (r, _N), 1)

        def tcount(j):
            m = jnp.logical_and(tie, col < j)
            ind = jnp.where(m, jnp.float32(1.0), jnp.float32(0.0))
            return jnp.sum(ind, axis=1, keepdims=True)

        def tbody(i, st):
            tlo, thi = st
            mid = tlo + (thi - tlo) // 2
            c = tcount(mid)
            small = c <= want
            tlo = jnp.where(small, mid, tlo)
            thi = jnp.where(small, thi, mid)
            return tlo, thi

        jlo0 = jnp.zeros((r, 1), jnp.int32)
        jhi0 = jnp.full((r, 1), _N + 1, jnp.int32)
        jcut, _ = jax.lax.fori_loop(0, 16, tbody, (jlo0, jhi0))

        ok_tie = jnp.logical_or(z >= lo + 1,
                                jnp.logical_and(tie, col < jcut))
        keep = jnp.logical_or(jnp.logical_and(surplus, ok_tie),
                              jnp.logical_and(jnp.logical_not(surplus),
                                              z >= lo))
        o_ref[...] = jnp.where(keep, x, jnp.float32(0.0))


def kernel(x):
    grid = _ROWS // _BLOCK_ROWS
    return pl.pallas_call(
        _topk_mask_body,
        grid=(grid,),
        in_specs=[pl.BlockSpec((_BLOCK_ROWS, _N), lambda i: (i, 0))],
        out_specs=pl.BlockSpec((_BLOCK_ROWS, _N), lambda i: (i, 0)),
        out_shape=jax.ShapeDtypeStruct((_ROWS, _N), jnp.float32),
    )(x)


# speedup vs baseline: 13.1183x; 1.3019x over previous
import jax
import jax.numpy as jnp
from jax.experimental import pallas as pl

_K = 64
_N = 32768
_ROWS = 128
_BLOCK_ROWS = 16


def _topk_mask_body(x_ref, o_ref):
    x = x_ref[...]
    r = x.shape[0]
    xi = jax.lax.bitcast_convert_type(x, jnp.int32)
    z = xi ^ ((xi >> 31) & jnp.int32(0x7FFFFFFF))

    def count_ge(t):
        ind = jnp.where(z >= t, jnp.float32(1.0), jnp.float32(0.0))
        return jnp.sum(ind, axis=1, keepdims=True)

    kf = jnp.float32(_K)
    one = jnp.full((r, 1), 1, jnp.int32)
    c1 = count_ge(one)

    zc = jnp.max(z.reshape(r, _N // 128, 128), axis=1)
    zmax = jnp.max(zc, axis=1, keepdims=True)

    def countc_ge(t):
        ind = jnp.where(zc >= t, jnp.float32(1.0), jnp.float32(0.0))
        return jnp.sum(ind, axis=1, keepdims=True)

    hi0 = jnp.maximum(zmax + 1, one + 1)

    def s1body(i, st):
        lo, hi = st
        q = (hi - lo) // 4
        m1 = lo + q
        m2 = lo + 2 * q
        m3 = lo + 3 * q
        m2 = jnp.maximum(m2, lo + 1)
        m3 = jnp.maximum(m3, m2)
        cc1 = countc_ge(m1)
        cc2 = countc_ge(m2)
        cc3 = countc_ge(m3)
        lo = jnp.where(cc1 >= kf, m1, lo)
        lo = jnp.where(cc2 >= kf, m2, lo)
        lo = jnp.where(cc3 >= kf, m3, lo)
        hi = jnp.where(cc3 < kf, m3, hi)
        hi = jnp.where(cc2 < kf, m2, hi)
        hi = jnp.where(cc1 < kf, m1, hi)
        return lo, hi

    s1lo, _ = jax.lax.fori_loop(0, 9, s1body, (one, hi0))

    done0 = jnp.where(c1 <= kf, jnp.int32(1), jnp.int32(0))
    lo0 = jnp.maximum(s1lo, one)

    def cond(state):
        lo, hi, done = state
        return jnp.min(done) < 1

    def body(state):
        lo, hi, done = state
        q = (hi - lo) // 4
        m1 = lo + q
        m2 = lo + 2 * q
        m3 = lo + 3 * q
        m2 = jnp.maximum(m2, lo + 1)
        m3 = jnp.maximum(m3, m2)
        c1_ = count_ge(m1)
        c2_ = count_ge(m2)
        c3_ = count_ge(m3)
        not_done = done < 1
        lo = jnp.where(jnp.logical_and(not_done, c1_ >= kf), m1, lo)
        lo = jnp.where(jnp.logical_and(not_done, c2_ >= kf), m2, lo)
        lo = jnp.where(jnp.logical_and(not_done, c3_ >= kf), m3, lo)
        hi = jnp.where(jnp.logical_and(not_done, c3_ < kf), m3, hi)
        hi = jnp.where(jnp.logical_and(not_done, c2_ < kf), m2, hi)
        hi = jnp.where(jnp.logical_and(not_done, c1_ < kf), m1, hi)
        lo = jnp.where(jnp.logical_and(not_done, c1_ == kf), m1, lo)
        lo = jnp.where(jnp.logical_and(not_done, c2_ == kf), m2, lo)
        lo = jnp.where(jnp.logical_and(not_done, c3_ == kf), m3, lo)
        hit = jnp.logical_or(jnp.logical_or(c1_ == kf, c2_ == kf), c3_ == kf)
        fin = jnp.logical_or(hit, hi - lo <= 1)
        done = jnp.maximum(done, jnp.where(fin, jnp.int32(1), jnp.int32(0)))
        return lo, hi, done

    lo, _, _ = jax.lax.while_loop(cond, body, (lo0, hi0, done0))

    ind = jnp.where(z >= lo, jnp.float32(1.0), jnp.float32(0.0))
    cnt = jnp.sum(ind, axis=1, keepdims=True)
    o_ref[...] = x * ind
    ties = jnp.any(cnt > kf)

    @pl.when(ties)
    def _():
        surplus = cnt > kf
        cgt = count_ge(lo + 1)
        want = jnp.where(surplus, kf - cgt, jnp.float32(_N))
        tie = jnp.logical_and(z == lo, surplus)
        col = jax.lax.broadcasted_iota(jnp.int32, (r, _N), 1)

        def tcount(j):
            m = jnp.logical_and(tie, col < j)
            ind = jnp.where(m, jnp.float32(1.0), jnp.float32(0.0))
            return jnp.sum(ind, axis=1, keepdims=True)

        def tbody(i, st):
            tlo, thi = st
            mid = tlo + (thi - tlo) // 2
            c = tcount(mid)
            small = c <= want
            tlo = jnp.where(small, mid, tlo)
            thi = jnp.where(small, thi, mid)
            return tlo, thi

        jlo0 = jnp.zeros((r, 1), jnp.int32)
        jhi0 = jnp.full((r, 1), _N + 1, jnp.int32)
        jcut, _ = jax.lax.fori_loop(0, 16, tbody, (jlo0, jhi0))

        ok_tie = jnp.logical_or(z >= lo + 1,
                                jnp.logical_and(tie, col < jcut))
        keep = jnp.logical_or(jnp.logical_and(surplus, ok_tie),
                              jnp.logical_and(jnp.logical_not(surplus),
                                              z >= lo))
        o_ref[...] = jnp.where(keep, x, jnp.float32(0.0))


def kernel(x):
    grid = _ROWS // _BLOCK_ROWS
    return pl.pallas_call(
        _topk_mask_body,
        grid=(grid,),
        in_specs=[pl.BlockSpec((_BLOCK_ROWS, _N), lambda i: (i, 0))],
        out_specs=pl.BlockSpec((_BLOCK_ROWS, _N), lambda i: (i, 0)),
        out_shape=jax.ShapeDtypeStruct((_ROWS, _N), jnp.float32),
    )(x)


# speedup vs baseline: 14.0480x; 1.0709x over previous
import jax
import jax.numpy as jnp
from jax.experimental import pallas as pl

_K = 64
_N = 32768
_ROWS = 128
_BLOCK_ROWS = 32


def _topk_mask_body(x_ref, o_ref):
    x = x_ref[...]
    r = x.shape[0]
    xi = jax.lax.bitcast_convert_type(x, jnp.int32)
    z = xi ^ ((xi >> 31) & jnp.int32(0x7FFFFFFF))

    def count_ge(t):
        ind = jnp.where(z >= t, jnp.float32(1.0), jnp.float32(0.0))
        return jnp.sum(ind, axis=1, keepdims=True)

    kf = jnp.float32(_K)
    one = jnp.full((r, 1), 1, jnp.int32)
    c1 = count_ge(one)

    zc = jnp.max(z.reshape(r, _N // 128, 128), axis=1)
    zmax = jnp.max(zc, axis=1, keepdims=True)

    def countc_ge(t):
        ind = jnp.where(zc >= t, jnp.float32(1.0), jnp.float32(0.0))
        return jnp.sum(ind, axis=1, keepdims=True)

    hi0 = jnp.maximum(zmax + 1, one + 1)

    def s1body(i, st):
        lo, hi = st
        q = (hi - lo) // 4
        m1 = lo + q
        m2 = lo + 2 * q
        m3 = lo + 3 * q
        m2 = jnp.maximum(m2, lo + 1)
        m3 = jnp.maximum(m3, m2)
        cc1 = countc_ge(m1)
        cc2 = countc_ge(m2)
        cc3 = countc_ge(m3)
        lo = jnp.where(cc1 >= kf, m1, lo)
        lo = jnp.where(cc2 >= kf, m2, lo)
        lo = jnp.where(cc3 >= kf, m3, lo)
        hi = jnp.where(cc3 < kf, m3, hi)
        hi = jnp.where(cc2 < kf, m2, hi)
        hi = jnp.where(cc1 < kf, m1, hi)
        return lo, hi

    s1lo, _ = jax.lax.fori_loop(0, 9, s1body, (one, hi0))

    done0 = jnp.where(c1 <= kf, jnp.int32(1), jnp.int32(0))
    lo0 = jnp.maximum(s1lo, one)

    def cond(state):
        lo, hi, done = state
        return jnp.min(done) < 1

    def body(state):
        lo, hi, done = state
        q = (hi - lo) // 4
        m1 = lo + q
        m2 = lo + 2 * q
        m3 = lo + 3 * q
        m2 = jnp.maximum(m2, lo + 1)
        m3 = jnp.maximum(m3, m2)
        c1_ = count_ge(m1)
        c2_ = count_ge(m2)
        c3_ = count_ge(m3)
        not_done = done < 1
        lo = jnp.where(jnp.logical_and(not_done, c1_ >= kf), m1, lo)
        lo = jnp.where(jnp.logical_and(not_done, c2_ >= kf), m2, lo)
        lo = jnp.where(jnp.logical_and(not_done, c3_ >= kf), m3, lo)
        hi = jnp.where(jnp.logical_and(not_done, c3_ < kf), m3, hi)
        hi = jnp.where(jnp.logical_and(not_done, c2_ < kf), m2, hi)
        hi = jnp.where(jnp.logical_and(not_done, c1_ < kf), m1, hi)
        lo = jnp.where(jnp.logical_and(not_done, c1_ == kf), m1, lo)
        lo = jnp.where(jnp.logical_and(not_done, c2_ == kf), m2, lo)
        lo = jnp.where(jnp.logical_and(not_done, c3_ == kf), m3, lo)
        hit = jnp.logical_or(jnp.logical_or(c1_ == kf, c2_ == kf), c3_ == kf)
        fin = jnp.logical_or(hit, hi - lo <= 1)
        done = jnp.maximum(done, jnp.where(fin, jnp.int32(1), jnp.int32(0)))
        return lo, hi, done

    lo, _, _ = jax.lax.while_loop(cond, body, (lo0, hi0, done0))

    ind = jnp.where(z >= lo, jnp.float32(1.0), jnp.float32(0.0))
    cnt = jnp.sum(ind, axis=1, keepdims=True)
    o_ref[...] = x * ind
    ties = jnp.any(cnt > kf)

    @pl.when(ties)
    def _():
        surplus = cnt > kf
        cgt = count_ge(lo + 1)
        want = jnp.where(surplus, kf - cgt, jnp.float32(_N))
        tie = jnp.logical_and(z == lo, surplus)
        col = jax.lax.broadcasted_iota(jnp.int32, (r, _N), 1)

        def tcount(j):
            m = jnp.logical_and(tie, col < j)
            ind = jnp.where(m, jnp.float32(1.0), jnp.float32(0.0))
            return jnp.sum(ind, axis=1, keepdims=True)

        def tbody(i, st):
            tlo, thi = st
            mid = tlo + (thi - tlo) // 2
            c = tcount(mid)
            small = c <= want
            tlo = jnp.where(small, mid, tlo)
            thi = jnp.where(small, thi, mid)
            return tlo, thi

        jlo0 = jnp.zeros((r, 1), jnp.int32)
        jhi0 = jnp.full((r, 1), _N + 1, jnp.int32)
        jcut, _ = jax.lax.fori_loop(0, 16, tbody, (jlo0, jhi0))

        ok_tie = jnp.logical_or(z >= lo + 1,
                                jnp.logical_and(tie, col < jcut))
        keep = jnp.logical_or(jnp.logical_and(surplus, ok_tie),
                              jnp.logical_and(jnp.logical_not(surplus),
                                              z >= lo))
        o_ref[...] = jnp.where(keep, x, jnp.float32(0.0))


def kernel(x):
    grid = _ROWS // _BLOCK_ROWS
    return pl.pallas_call(
        _topk_mask_body,
        grid=(grid,),
        in_specs=[pl.BlockSpec((_BLOCK_ROWS, _N), lambda i: (i, 0))],
        out_specs=pl.BlockSpec((_BLOCK_ROWS, _N), lambda i: (i, 0)),
        out_shape=jax.ShapeDtypeStruct((_ROWS, _N), jnp.float32),
    )(x)
